# Initial kernel scaffold; baseline (speedup 1.0000x reference)
import functools

import jax
import jax.numpy as jnp
from jax import lax
from jax.experimental import pallas as pl
from jax.experimental.pallas import tpu as pltpu
from jax.experimental.pallas import tpu_sc as plsc

BATCH = 4096
SEQ = 200
DIM = 64
N = BATCH * SEQ

NUM_CORES = 2
NUM_SUBCORES = 16
NW = NUM_CORES * NUM_SUBCORES
ROWS_PER_W = N // NW

GB = 128
KSUB = 10
CHUNK = GB * KSUB
G = ROWS_PER_W // CHUNK

LANES = 16
JD = DIM // LANES


def _body(xf_hbm, table_hbm, pos_hbm, out_hbm, idx_v, buf, pos_t, sem):
    wid = lax.axis_index("s") * NUM_CORES + lax.axis_index("c")
    pltpu.sync_copy(pos_hbm.at[pl.ds(0, SEQ)], pos_t)
    base0 = wid * ROWS_PER_W

    def chunk_body(g, carry):
        base = base0 + g * CHUNK
        pltpu.sync_copy(xf_hbm.at[pl.ds(base // GB, KSUB)], idx_v)
        copies = [
            pltpu.async_copy(
                table_hbm.at[idx_v.at[k]],
                buf.at[pl.ds(k * GB, GB)],
                sem,
            )
            for k in range(KSUB)
        ]
        for c in copies:
            c.wait()

        off = lax.rem(base, SEQ)

        def add_body(i, pr):
            for j in range(JD):
                v = pos_t[pr, pl.ds(j * LANES, LANES)]
                plsc.addupdate(buf.at[i, pl.ds(j * LANES, LANES)], v)
            return lax.select(pr == SEQ - 1, 0, pr + 1)

        lax.fori_loop(0, CHUNK, add_body, off)

        pltpu.sync_copy(buf, out_hbm.at[pl.ds(base, CHUNK)])
        return carry

    lax.fori_loop(0, G, chunk_body, 0)


@jax.jit
def _run(xf, table, pos):
    mesh = plsc.VectorSubcoreMesh(core_axis_name="c", subcore_axis_name="s")
    f = pl.kernel(
        _body,
        out_type=jax.ShapeDtypeStruct((N, DIM), jnp.float32),
        mesh=mesh,
        scratch_types=[
            pltpu.VMEM((KSUB, GB), jnp.int32),
            pltpu.VMEM((CHUNK, DIM), jnp.float32),
            pltpu.VMEM((SEQ, DIM), jnp.float32),
            pltpu.SemaphoreType.DMA,
        ],
    )
    return f(xf, table, pos)


def kernel(x, table, pos_encoding):
    xf = x.reshape(N // GB, GB).astype(jnp.int32)
    out = _run(xf, table, pos_encoding)
    return out.reshape(BATCH, SEQ, DIM)


# speedup vs baseline: 2.7554x; 2.7554x over previous
import functools

import jax
import jax.numpy as jnp
from jax import lax
from jax.experimental import pallas as pl
from jax.experimental.pallas import tpu as pltpu
from jax.experimental.pallas import tpu_sc as plsc

BATCH = 4096
SEQ = 200
DIM = 64
N = BATCH * SEQ

NUM_CORES = 2
NUM_SUBCORES = 16
NW = NUM_CORES * NUM_SUBCORES
ROWS_PER_W = N // NW

GB = 128
KSUB = 8
CHUNK = GB * KSUB
G = ROWS_PER_W // CHUNK

LANES = 16
JD = DIM // LANES


def _body(xf_hbm, table_hbm, pos_hbm, out_hbm, idx_v, buf, pos_t, sem):
    wid = lax.axis_index("s") * NUM_CORES + lax.axis_index("c")
    pltpu.sync_copy(pos_hbm.at[pl.ds(0, SEQ)], pos_t)
    base0 = wid * ROWS_PER_W

    def chunk_body(g, carry):
        base = pl.multiple_of(base0 + g * CHUNK, CHUNK)
        pltpu.sync_copy(xf_hbm.at[pl.ds(pl.multiple_of(base // GB, KSUB), KSUB)], idx_v)
        copies = [
            pltpu.async_copy(
                table_hbm.at[idx_v.at[k]],
                buf.at[pl.ds(k * GB, GB)],
                sem,
            )
            for k in range(KSUB)
        ]
        for c in copies:
            c.wait()

        off = lax.rem(base, SEQ)

        def add_body(i, pr):
            for j in range(JD):
                v = pos_t[pr, pl.ds(j * LANES, LANES)]
                plsc.addupdate(buf.at[i, pl.ds(j * LANES, LANES)], v)
            return lax.select(pr == SEQ - 1, 0, pr + 1)

        lax.fori_loop(0, CHUNK, add_body, off)

        pltpu.sync_copy(buf, out_hbm.at[pl.ds(base, CHUNK)])
        return carry

    lax.fori_loop(0, G, chunk_body, 0)


@jax.jit
def _run(xf, table, pos):
    mesh = plsc.VectorSubcoreMesh(core_axis_name="c", subcore_axis_name="s")
    f = pl.kernel(
        _body,
        out_type=jax.ShapeDtypeStruct((N, DIM), jnp.float32),
        mesh=mesh,
        compiler_params=pltpu.CompilerParams(use_tc_tiling_on_sc=False),
        scratch_types=[
            pltpu.VMEM((KSUB, GB), jnp.int32),
            pltpu.VMEM((CHUNK, DIM), jnp.float32),
            pltpu.VMEM((SEQ, DIM), jnp.float32),
            pltpu.SemaphoreType.DMA,
        ],
    )
    return f(xf, table, pos)


def kernel(x, table, pos_encoding):
    xf = x.reshape(N // GB, GB).astype(jnp.int32)
    out = _run(xf, table, pos_encoding)
    return out.reshape(BATCH, SEQ, DIM)


# speedup vs baseline: 3.4012x; 1.2344x over previous
import jax
import jax.numpy as jnp
from jax import lax
from jax.experimental import pallas as pl
from jax.experimental.pallas import tpu as pltpu
from jax.experimental.pallas import tpu_sc as plsc

BATCH = 4096
SEQ = 200
DIM = 64
N = BATCH * SEQ

NUM_CORES = 2
NUM_SUBCORES = 16
NW = NUM_CORES * NUM_SUBCORES
ROWS_PER_W = N // NW

GB = 128
KSUB = 8
CHUNK = GB * KSUB
G = ROWS_PER_W // CHUNK

TMPL = SEQ + CHUNK
TMPL = (TMPL + 7) // 8 * 8


def _body(xf_hbm, table_hbm, tmpl_hbm, out_hbm, idx_v, buf, sem):
    wid = lax.axis_index("s") * NUM_CORES + lax.axis_index("c")
    base0 = wid * ROWS_PER_W

    def chunk_body(g, carry):
        base = pl.multiple_of(base0 + g * CHUNK, CHUNK)
        pltpu.sync_copy(
            xf_hbm.at[pl.ds(pl.multiple_of(base // GB, KSUB), KSUB)], idx_v
        )
        off = pl.multiple_of(lax.rem(g * CHUNK, SEQ), 8)
        pltpu.sync_copy(tmpl_hbm.at[pl.ds(off, CHUNK)], buf)
        copies = [
            pltpu.async_copy(
                table_hbm.at[idx_v.at[k]],
                buf.at[pl.ds(k * GB, GB)],
                sem,
                add=True,
            )
            for k in range(KSUB)
        ]
        for c in copies:
            c.wait()

        pltpu.sync_copy(buf, out_hbm.at[pl.ds(base, CHUNK)])
        return carry

    lax.fori_loop(0, G, chunk_body, 0)


@jax.jit
def _run(xf, table, tmpl):
    mesh = plsc.VectorSubcoreMesh(core_axis_name="c", subcore_axis_name="s")
    f = pl.kernel(
        _body,
        out_type=jax.ShapeDtypeStruct((N, DIM), jnp.float32),
        mesh=mesh,
        compiler_params=pltpu.CompilerParams(use_tc_tiling_on_sc=False),
        scratch_types=[
            pltpu.VMEM((KSUB, GB), jnp.int32),
            pltpu.VMEM((CHUNK, DIM), jnp.float32),
            pltpu.SemaphoreType.DMA,
        ],
    )
    return f(xf, table, tmpl)


def kernel(x, table, pos_encoding):
    xf = x.reshape(N // GB, GB).astype(jnp.int32)
    reps = -(-TMPL // SEQ)
    tmpl = jnp.tile(pos_encoding[:SEQ], (reps, 1))[:TMPL]
    out = _run(xf, table, tmpl)
    return out.reshape(BATCH, SEQ, DIM)
